# Initial kernel scaffold; baseline (speedup 1.0000x reference)
import jax
import jax.numpy as jnp
from jax.experimental import pallas as pl


def kernel(node_input, edge_attr, edge_scalars, edge_src, edge_dst, batch, w_exp, b_exp, rad_w1, rad_b1, ln_g, ln_b, rad_w2, rad_offset, w_proj, b_proj):
    raise NotImplementedError("write your pallas kernel here")


# speedup vs baseline: 2.9940x; 2.9940x over previous
import functools
import math

import jax
import jax.numpy as jnp
from jax import lax
from jax.experimental import pallas as pl
from jax.experimental.pallas import tpu as pltpu
from jax.experimental.pallas import tpu_sc as plsc


def _edge_mlp_kernel(es_ref, attr_ref, w1_ref, b1_ref, g_ref, lb_ref, out_ref,
                     *, fc, d):
    h = jnp.dot(es_ref[...], w1_ref[...], preferred_element_type=jnp.float32)
    h = h + b1_ref[...]
    mu = jnp.mean(h, axis=-1, keepdims=True)
    var = jnp.mean(jnp.square(h - mu), axis=-1, keepdims=True)
    h = (h - mu) * lax.rsqrt(var + 1e-5) * g_ref[...] + lb_ref[...]
    h = h * jax.nn.sigmoid(h)
    a = attr_ref[...]
    s = h * a
    be = s.shape[0]
    ones = jnp.ones_like(a)
    pad = jnp.zeros((be, d - fc - 2), jnp.float32)
    out_ref[...] = jnp.concatenate([s, a, ones, pad], axis=-1)


def _edge_stage(edge_scalars, edge_attr, rad_w1, rad_b1, ln_g, ln_b, d):
    e, fc = edge_scalars.shape
    be = 2000
    assert e % be == 0
    grid = e // be
    return pl.pallas_call(
        functools.partial(_edge_mlp_kernel, fc=fc, d=d),
        grid=(grid,),
        in_specs=[
            pl.BlockSpec((be, fc), lambda i: (i, 0)),
            pl.BlockSpec((be, 1), lambda i: (i, 0)),
            pl.BlockSpec((fc, fc), lambda i: (0, 0)),
            pl.BlockSpec((fc,), lambda i: (0,)),
            pl.BlockSpec((fc,), lambda i: (0,)),
            pl.BlockSpec((fc,), lambda i: (0,)),
        ],
        out_specs=pl.BlockSpec((be, d), lambda i: (i, 0)),
        out_shape=jax.ShapeDtypeStruct((e, d), jnp.float32),
    )(edge_scalars, edge_attr, rad_w1, rad_b1, ln_g, ln_b)


def _sc_scatter_kernel(payload_hbm, dst_hbm, zeros_hbm, out_hbm,
                       pbuf, ibuf, pbuf_t, ibuf_t, acc,
                       *, n, per_tile, n_per_tile, chunk, tail):
    c = lax.axis_index("c")
    s = lax.axis_index("s")
    pltpu.sync_copy(zeros_hbm, acc.at[pl.ds(s * n_per_tile, n_per_tile), :])
    plsc.subcore_barrier()
    base = (c * 16 + s) * per_tile

    def body(g, _):
        off = base + g * chunk
        pltpu.sync_copy(dst_hbm.at[pl.ds(off, chunk)], ibuf)
        pltpu.sync_copy(payload_hbm.at[pl.ds(off, chunk), :], pbuf)
        pltpu.sync_copy(pbuf, acc.at[ibuf], add=True)
        return 0

    lax.fori_loop(0, per_tile // chunk, body, 0)
    if tail:
        off = base + (per_tile // chunk) * chunk
        pltpu.sync_copy(dst_hbm.at[pl.ds(off, tail)], ibuf_t)
        pltpu.sync_copy(payload_hbm.at[pl.ds(off, tail), :], pbuf_t)
        pltpu.sync_copy(pbuf_t, acc.at[ibuf_t], add=True)
    plsc.subcore_barrier()
    pltpu.sync_copy(acc.at[pl.ds(s * n_per_tile, n_per_tile), :],
                    out_hbm.at[c, pl.ds(s * n_per_tile, n_per_tile), :])


def _scatter_stage(payload, edge_dst, n):
    e, d = payload.shape
    assert e % 32 == 0
    per_tile = e // 32
    chunk = 128
    tail = per_tile % chunk
    n_pad = ((n + 127) // 128) * 128
    n_per_tile = n_pad // 16
    zeros = jnp.zeros((n_per_tile, d), jnp.float32)
    mesh = plsc.VectorSubcoreMesh(core_axis_name="c", subcore_axis_name="s")
    scratch = [
        pltpu.VMEM((chunk, d), jnp.float32),
        pltpu.VMEM((chunk,), jnp.int32),
        pltpu.VMEM((max(tail, 8), d), jnp.float32),
        pltpu.VMEM((max(tail, 8),), jnp.int32),
        pltpu.VMEM_SHARED((n_pad, d), jnp.float32),
    ]
    kern = pl.kernel(
        functools.partial(_sc_scatter_kernel, n=n_pad, per_tile=per_tile,
                          n_per_tile=n_per_tile, chunk=chunk, tail=tail),
        out_type=jax.ShapeDtypeStruct((2, n_pad, d), jnp.float32),
        mesh=mesh,
        scratch_types=scratch,
    )
    return kern(payload, edge_dst, zeros)


def _proj_kernel(part_ref, wexp_ref, bexp_ref, w2_ref, roff_ref, wproj_ref,
                 bproj_ref, out_ref, *, fc, d, avg_agg):
    p = part_ref[0] + part_ref[1]
    s = p[:, :fc]
    a = p[:, fc:fc + 1]
    cnt = p[:, fc + 1:fc + 2]
    v = wexp_ref[...][0] + bexp_ref[...]
    wc = jnp.dot(w2_ref[...] * v[None, :], wproj_ref[...],
                 preferred_element_type=jnp.float32)
    rvec = jnp.dot((v * roff_ref[...])[None, :], wproj_ref[...],
                   preferred_element_type=jnp.float32)
    scale = 1.0 / (math.sqrt(d) * math.sqrt(avg_agg))
    out = (jnp.dot(s, wc, preferred_element_type=jnp.float32) + a * rvec) * scale
    out = out + cnt * (bproj_ref[...][None, :] / math.sqrt(avg_agg))
    out_ref[...] = out


def _proj_stage(partials, n, w_exp, b_exp, rad_w2, rad_offset, w_proj, b_proj,
                avg_agg):
    _, _, d = partials.shape
    fc = rad_w2.shape[0]
    bn = 2000
    assert n % bn == 0
    grid = n // bn
    return pl.pallas_call(
        functools.partial(_proj_kernel, fc=fc, d=d, avg_agg=avg_agg),
        grid=(grid,),
        in_specs=[
            pl.BlockSpec((2, bn, d), lambda i: (0, i, 0)),
            pl.BlockSpec((1, d), lambda i: (0, 0)),
            pl.BlockSpec((d,), lambda i: (0,)),
            pl.BlockSpec((fc, d), lambda i: (0, 0)),
            pl.BlockSpec((d,), lambda i: (0,)),
            pl.BlockSpec((d, d), lambda i: (0, 0)),
            pl.BlockSpec((d,), lambda i: (0,)),
        ],
        out_specs=pl.BlockSpec((bn, d), lambda i: (i, 0)),
        out_shape=jax.ShapeDtypeStruct((n, d), jnp.float32),
    )(partials, w_exp, b_exp, rad_w2, rad_offset, w_proj, b_proj)


def kernel(node_input, edge_attr, edge_scalars, edge_src, edge_dst, batch,
           w_exp, b_exp, rad_w1, rad_b1, ln_g, ln_b, rad_w2, rad_offset,
           w_proj, b_proj):
    n = node_input.shape[0]
    d = node_input.shape[1]
    avg_agg = 32.0
    payload = _edge_stage(edge_scalars, edge_attr, rad_w1, rad_b1, ln_g, ln_b, d)
    partials = _scatter_stage(payload, edge_dst, n)
    return _proj_stage(partials, n, w_exp, b_exp, rad_w2, rad_offset, w_proj,
                       b_proj, avg_agg)


# speedup vs baseline: 3.5216x; 1.1762x over previous
import functools
import math

import jax
import jax.numpy as jnp
from jax import lax
from jax.experimental import pallas as pl
from jax.experimental.pallas import tpu as pltpu
from jax.experimental.pallas import tpu_sc as plsc


def _edge_mlp_kernel(es_ref, attr_ref, w1_ref, b1_ref, g_ref, lb_ref, out_ref,
                     *, fc, d):
    h = jnp.dot(es_ref[...], w1_ref[...], preferred_element_type=jnp.float32)
    h = h + b1_ref[...]
    mu = jnp.mean(h, axis=-1, keepdims=True)
    var = jnp.mean(jnp.square(h - mu), axis=-1, keepdims=True)
    h = (h - mu) * lax.rsqrt(var + 1e-5) * g_ref[...] + lb_ref[...]
    h = h * jax.nn.sigmoid(h)
    a = attr_ref[...]
    s = h * a
    be = s.shape[0]
    ones = jnp.ones_like(a)
    pad = jnp.zeros((be, d - fc - 2), jnp.float32)
    out_ref[...] = jnp.concatenate([s, a, ones, pad], axis=-1)


def _edge_stage(edge_scalars, edge_attr, rad_w1, rad_b1, ln_g, ln_b, d):
    e, fc = edge_scalars.shape
    be = 2000
    assert e % be == 0
    grid = e // be
    return pl.pallas_call(
        functools.partial(_edge_mlp_kernel, fc=fc, d=d),
        grid=(grid,),
        in_specs=[
            pl.BlockSpec((be, fc), lambda i: (i, 0)),
            pl.BlockSpec((be, 1), lambda i: (i, 0)),
            pl.BlockSpec((fc, fc), lambda i: (0, 0)),
            pl.BlockSpec((fc,), lambda i: (0,)),
            pl.BlockSpec((fc,), lambda i: (0,)),
            pl.BlockSpec((fc,), lambda i: (0,)),
        ],
        out_specs=pl.BlockSpec((be, d), lambda i: (i, 0)),
        out_shape=jax.ShapeDtypeStruct((e, d), jnp.float32),
    )(edge_scalars, edge_attr, rad_w1, rad_b1, ln_g, ln_b)


def _sc_scatter_kernel(payload_hbm, dst_hbm, zeros_hbm, out_hbm,
                       pbuf, ibuf, pbuf_t, ibuf_t, acc, psem0, psem1,
                       isem0, isem1,
                       *, n, per_tile, n_per_tile, chunk, tail):
    c = lax.axis_index("c")
    s = lax.axis_index("s")
    pltpu.sync_copy(zeros_hbm, acc.at[pl.ds(s * n_per_tile, n_per_tile), :])
    plsc.subcore_barrier()
    base = (c * 16 + s) * per_tile
    units = per_tile // chunk
    assert units % 2 == 0
    psems = (psem0, psem1)
    isems = (isem0, isem1)

    def start_load(u, slot):
        off = base + u * chunk
        pltpu.async_copy(dst_hbm.at[pl.ds(off, chunk)], ibuf.at[slot],
                         isems[slot])
        pltpu.async_copy(payload_hbm.at[pl.ds(off, chunk), :], pbuf.at[slot],
                         psems[slot])

    def wait_scatter(slot):
        pltpu.make_async_copy(dst_hbm.at[pl.ds(0, chunk)], ibuf.at[slot],
                              isems[slot]).wait()
        pltpu.make_async_copy(payload_hbm.at[pl.ds(0, chunk), :],
                              pbuf.at[slot], psems[slot]).wait()
        pltpu.sync_copy(pbuf.at[slot], acc.at[ibuf.at[slot]], add=True)

    start_load(0, 0)

    def body(g, _):
        u = 2 * g
        start_load(u + 1, 1)
        wait_scatter(0)

        @pl.when(u + 2 < units)
        def _():
            start_load(u + 2, 0)

        wait_scatter(1)
        return 0

    lax.fori_loop(0, units // 2, body, 0)
    if tail:
        off = base + units * chunk
        pltpu.sync_copy(dst_hbm.at[pl.ds(off, tail)], ibuf_t)
        pltpu.sync_copy(payload_hbm.at[pl.ds(off, tail), :], pbuf_t)
        pltpu.sync_copy(pbuf_t, acc.at[ibuf_t], add=True)
    plsc.subcore_barrier()
    pltpu.sync_copy(acc.at[pl.ds(s * n_per_tile, n_per_tile), :],
                    out_hbm.at[c, pl.ds(s * n_per_tile, n_per_tile), :])


def _scatter_stage(payload, edge_dst, n):
    e, d = payload.shape
    assert e % 32 == 0
    per_tile = e // 32
    chunk = 128
    tail = per_tile % chunk
    n_pad = ((n + 127) // 128) * 128
    n_per_tile = n_pad // 16
    zeros = jnp.zeros((n_per_tile, d), jnp.float32)
    mesh = plsc.VectorSubcoreMesh(core_axis_name="c", subcore_axis_name="s")
    scratch = [
        pltpu.VMEM((2, chunk, d), jnp.float32),
        pltpu.VMEM((2, chunk), jnp.int32),
        pltpu.VMEM((max(tail, 8), d), jnp.float32),
        pltpu.VMEM((max(tail, 8),), jnp.int32),
        pltpu.VMEM_SHARED((n_pad, d), jnp.float32),
        pltpu.SemaphoreType.DMA,
        pltpu.SemaphoreType.DMA,
        pltpu.SemaphoreType.DMA,
        pltpu.SemaphoreType.DMA,
    ]
    kern = pl.kernel(
        functools.partial(_sc_scatter_kernel, n=n_pad, per_tile=per_tile,
                          n_per_tile=n_per_tile, chunk=chunk, tail=tail),
        out_type=jax.ShapeDtypeStruct((2, n_pad, d), jnp.float32),
        mesh=mesh,
        scratch_types=scratch,
    )
    return kern(payload, edge_dst, zeros)


def _proj_kernel(part_ref, wexp_ref, bexp_ref, w2_ref, roff_ref, wproj_ref,
                 bproj_ref, out_ref, *, fc, d, avg_agg):
    p = part_ref[0] + part_ref[1]
    s = p[:, :fc]
    a = p[:, fc:fc + 1]
    cnt = p[:, fc + 1:fc + 2]
    v = wexp_ref[...][0] + bexp_ref[...]
    wc = jnp.dot(w2_ref[...] * v[None, :], wproj_ref[...],
                 preferred_element_type=jnp.float32)
    rvec = jnp.dot((v * roff_ref[...])[None, :], wproj_ref[...],
                   preferred_element_type=jnp.float32)
    scale = 1.0 / (math.sqrt(d) * math.sqrt(avg_agg))
    out = (jnp.dot(s, wc, preferred_element_type=jnp.float32) + a * rvec) * scale
    out = out + cnt * (bproj_ref[...][None, :] / math.sqrt(avg_agg))
    out_ref[...] = out


def _proj_stage(partials, n, w_exp, b_exp, rad_w2, rad_offset, w_proj, b_proj,
                avg_agg):
    _, _, d = partials.shape
    fc = rad_w2.shape[0]
    bn = 2000
    assert n % bn == 0
    grid = n // bn
    return pl.pallas_call(
        functools.partial(_proj_kernel, fc=fc, d=d, avg_agg=avg_agg),
        grid=(grid,),
        in_specs=[
            pl.BlockSpec((2, bn, d), lambda i: (0, i, 0)),
            pl.BlockSpec((1, d), lambda i: (0, 0)),
            pl.BlockSpec((d,), lambda i: (0,)),
            pl.BlockSpec((fc, d), lambda i: (0, 0)),
            pl.BlockSpec((d,), lambda i: (0,)),
            pl.BlockSpec((d, d), lambda i: (0, 0)),
            pl.BlockSpec((d,), lambda i: (0,)),
        ],
        out_specs=pl.BlockSpec((bn, d), lambda i: (i, 0)),
        out_shape=jax.ShapeDtypeStruct((n, d), jnp.float32),
    )(partials, w_exp, b_exp, rad_w2, rad_offset, w_proj, b_proj)


def kernel(node_input, edge_attr, edge_scalars, edge_src, edge_dst, batch,
           w_exp, b_exp, rad_w1, rad_b1, ln_g, ln_b, rad_w2, rad_offset,
           w_proj, b_proj):
    n = node_input.shape[0]
    d = node_input.shape[1]
    avg_agg = 32.0
    payload = _edge_stage(edge_scalars, edge_attr, rad_w1, rad_b1, ln_g, ln_b, d)
    partials = _scatter_stage(payload, edge_dst, n)
    return _proj_stage(partials, n, w_exp, b_exp, rad_w2, rad_offset, w_proj,
                       b_proj, avg_agg)


# speedup vs baseline: 8.2472x; 2.3419x over previous
import functools
import math

import jax
import jax.numpy as jnp
from jax import lax
from jax.experimental import pallas as pl
from jax.experimental.pallas import tpu as pltpu
from jax.experimental.pallas import tpu_sc as plsc


def _edge_mlp_kernel(esa_ref, esb_ref, aa_ref, ab_ref, wt_ref, bt_ref,
                     gt_ref, lbt_ref, mavg_ref, out_ref, *, fc):
    x = jnp.concatenate([esa_ref[...], esb_ref[...]], axis=0)
    h = jnp.dot(wt_ref[...], x, preferred_element_type=jnp.float32)
    h = h + bt_ref[...]
    mavg = mavg_ref[...]
    mu = jnp.dot(mavg, h, preferred_element_type=jnp.float32)
    msq = jnp.dot(mavg, h * h, preferred_element_type=jnp.float32)
    var = msq - mu * mu
    hn = (h - mu) * lax.rsqrt(var + 1e-5) * gt_ref[...] + lbt_ref[...]
    sil = hn * jax.nn.sigmoid(hn)
    bp = sil.shape[1]
    aa = aa_ref[...]
    ab = ab_ref[...]
    sa = sil[:fc, :] * jnp.broadcast_to(aa, (fc, bp))
    sb = sil[fc:, :] * jnp.broadcast_to(ab, (fc, bp))
    ones = jnp.ones((1, bp), jnp.float32)
    zpad = jnp.zeros((fc - 2, bp), jnp.float32)
    ta = jnp.concatenate([sa, aa, ones, zpad], axis=0)
    tb = jnp.concatenate([sb, ab, ones, zpad], axis=0)
    out_ref[0] = ta.T
    out_ref[1] = tb.T


def _edge_stage(edge_scalars, edge_attr, rad_w1, rad_b1, ln_g, ln_b):
    e, fc = edge_scalars.shape
    e2 = e // 2
    es_t = edge_scalars.T
    a_row = edge_attr.T
    z = jnp.zeros((fc, fc), jnp.float32)
    w1t = rad_w1.T
    wt = jnp.block([[w1t, z], [z, w1t]])
    ones_avg = jnp.full((fc, fc), 1.0 / fc, jnp.float32)
    mavg = jnp.block([[ones_avg, z], [z, ones_avg]])
    bt = jnp.concatenate([rad_b1, rad_b1])[:, None]
    gt = jnp.concatenate([ln_g, ln_g])[:, None]
    lbt = jnp.concatenate([ln_b, ln_b])[:, None]
    bp = 3200
    assert e2 % bp == 0
    grid = e2 // bp
    d2 = 2 * fc
    return pl.pallas_call(
        functools.partial(_edge_mlp_kernel, fc=fc),
        grid=(grid,),
        in_specs=[
            pl.BlockSpec((fc, bp), lambda i: (0, i)),
            pl.BlockSpec((fc, bp), lambda i, g=grid: (0, i + g)),
            pl.BlockSpec((1, bp), lambda i: (0, i)),
            pl.BlockSpec((1, bp), lambda i, g=grid: (0, i + g)),
            pl.BlockSpec((d2, d2), lambda i: (0, 0)),
            pl.BlockSpec((d2, 1), lambda i: (0, 0)),
            pl.BlockSpec((d2, 1), lambda i: (0, 0)),
            pl.BlockSpec((d2, 1), lambda i: (0, 0)),
            pl.BlockSpec((d2, d2), lambda i: (0, 0)),
        ],
        out_specs=pl.BlockSpec((2, bp, d2), lambda i: (0, i, 0)),
        out_shape=jax.ShapeDtypeStruct((2, e2, d2), jnp.float32),
    )(es_t, es_t, a_row, a_row, wt, bt, gt, lbt, mavg)


def _sc_scatter_kernel(payload_hbm, dst_hbm, zeros_hbm, out_hbm,
                       pbuf, ibuf, pbuf_t, ibuf_t, acc, psem0, psem1,
                       isem0, isem1,
                       *, e2, per_tile, n_per_tile, chunk, tail):
    c = lax.axis_index("c")
    s = lax.axis_index("s")
    pltpu.sync_copy(zeros_hbm, acc.at[pl.ds(s * n_per_tile, n_per_tile), :])
    plsc.subcore_barrier()
    base = s * per_tile
    units = per_tile // chunk
    assert units % 2 == 0
    psems = (psem0, psem1)
    isems = (isem0, isem1)

    def start_load(u, slot):
        off = base + u * chunk
        pltpu.async_copy(payload_hbm.at[c, pl.ds(off, chunk), :],
                         pbuf.at[slot], psems[slot])
        pltpu.async_copy(dst_hbm.at[pl.ds(c * e2 + off, chunk)],
                         ibuf.at[slot], isems[slot])

    def do_scatter(slot):
        pltpu.make_async_copy(payload_hbm.at[c, pl.ds(0, chunk), :],
                              pbuf.at[slot], psems[slot]).wait()
        pltpu.make_async_copy(dst_hbm.at[pl.ds(0, chunk)], ibuf.at[slot],
                              isems[slot]).wait()
        pltpu.sync_copy(pbuf.at[slot], acc.at[ibuf.at[slot]], add=True)

    start_load(0, 0)

    def body(g, _):
        u = 2 * g
        start_load(u + 1, 1)
        do_scatter(0)

        @pl.when(u + 2 < units)
        def _():
            start_load(u + 2, 0)

        do_scatter(1)
        return 0

    lax.fori_loop(0, units // 2, body, 0)
    if tail:
        off = base + units * chunk
        pltpu.sync_copy(payload_hbm.at[c, pl.ds(off, tail), :], pbuf_t)
        pltpu.sync_copy(dst_hbm.at[pl.ds(c * e2 + off, tail)], ibuf_t)
        pltpu.sync_copy(pbuf_t, acc.at[ibuf_t], add=True)
    plsc.subcore_barrier()
    pltpu.sync_copy(acc.at[pl.ds(s * n_per_tile, n_per_tile), :],
                    out_hbm.at[c, pl.ds(s * n_per_tile, n_per_tile), :])


def _scatter_stage(payload, edge_dst, n):
    _, e2, d2 = payload.shape
    assert e2 % 16 == 0
    per_tile = e2 // 16
    chunk = 128
    tail = per_tile % chunk
    n_pad = ((n + 127) // 128) * 128
    n_per_tile = n_pad // 16
    zeros = jnp.zeros((n_per_tile, d2), jnp.float32)
    mesh = plsc.VectorSubcoreMesh(core_axis_name="c", subcore_axis_name="s")
    scratch = [
        pltpu.VMEM((2, chunk, d2), jnp.float32),
        pltpu.VMEM((2, chunk), jnp.int32),
        pltpu.VMEM((max(tail, 8), d2), jnp.float32),
        pltpu.VMEM((max(tail, 8),), jnp.int32),
        pltpu.VMEM_SHARED((n_pad, d2), jnp.float32),
        pltpu.SemaphoreType.DMA,
        pltpu.SemaphoreType.DMA,
        pltpu.SemaphoreType.DMA,
        pltpu.SemaphoreType.DMA,
    ]
    kern = pl.kernel(
        functools.partial(_sc_scatter_kernel, e2=e2, per_tile=per_tile,
                          n_per_tile=n_per_tile, chunk=chunk, tail=tail),
        out_type=jax.ShapeDtypeStruct((2, n_pad, d2), jnp.float32),
        mesh=mesh,
        scratch_types=scratch,
    )
    return kern(payload, edge_dst, zeros)


def _proj_kernel(part_ref, wexp_ref, bexp_ref, w2_ref, roff_ref, wproj_ref,
                 bproj_ref, out_ref, *, fc, d, avg_agg):
    p = part_ref[0] + part_ref[1]
    s = p[:, :fc]
    a = p[:, fc:fc + 1]
    cnt = p[:, fc + 1:fc + 2]
    v = wexp_ref[...][0] + bexp_ref[...]
    wc = jnp.dot(w2_ref[...] * v[None, :], wproj_ref[...],
                 preferred_element_type=jnp.float32)
    rvec = jnp.dot((v * roff_ref[...])[None, :], wproj_ref[...],
                   preferred_element_type=jnp.float32)
    scale = 1.0 / (math.sqrt(d) * math.sqrt(avg_agg))
    out = (jnp.dot(s, wc, preferred_element_type=jnp.float32) + a * rvec) * scale
    out = out + cnt * (bproj_ref[...][None, :] / math.sqrt(avg_agg))
    out_ref[...] = out


def _proj_stage(partials, n, w_exp, b_exp, rad_w2, rad_offset, w_proj, b_proj,
                avg_agg):
    _, _, d = partials.shape
    fc = rad_w2.shape[0]
    bn = 2000
    assert n % bn == 0
    grid = n // bn
    return pl.pallas_call(
        functools.partial(_proj_kernel, fc=fc, d=d, avg_agg=avg_agg),
        grid=(grid,),
        in_specs=[
            pl.BlockSpec((2, bn, d), lambda i: (0, i, 0)),
            pl.BlockSpec((1, d), lambda i: (0, 0)),
            pl.BlockSpec((d,), lambda i: (0,)),
            pl.BlockSpec((fc, d), lambda i: (0, 0)),
            pl.BlockSpec((d,), lambda i: (0,)),
            pl.BlockSpec((d, d), lambda i: (0, 0)),
            pl.BlockSpec((d,), lambda i: (0,)),
        ],
        out_specs=pl.BlockSpec((bn, d), lambda i: (i, 0)),
        out_shape=jax.ShapeDtypeStruct((n, d), jnp.float32),
    )(partials, w_exp, b_exp, rad_w2, rad_offset, w_proj, b_proj)


def kernel(node_input, edge_attr, edge_scalars, edge_src, edge_dst, batch,
           w_exp, b_exp, rad_w1, rad_b1, ln_g, ln_b, rad_w2, rad_offset,
           w_proj, b_proj):
    n = node_input.shape[0]
    avg_agg = 32.0
    payload = _edge_stage(edge_scalars, edge_attr, rad_w1, rad_b1, ln_g, ln_b)
    partials = _scatter_stage(payload, edge_dst, n)
    return _proj_stage(partials, n, w_exp, b_exp, rad_w2, rad_offset, w_proj,
                       b_proj, avg_agg)


# speedup vs baseline: 8.5804x; 1.0404x over previous
import functools
import math

import jax
import jax.numpy as jnp
from jax import lax
from jax.experimental import pallas as pl
from jax.experimental.pallas import tpu as pltpu
from jax.experimental.pallas import tpu_sc as plsc


def _edge_mlp_kernel(esa_ref, esb_ref, aa_ref, ab_ref, wt_ref, bt_ref,
                     gt_ref, lbt_ref, mavg_ref, out_ref, *, fc):
    x = jnp.concatenate([esa_ref[...], esb_ref[...]], axis=0)
    h = jnp.dot(wt_ref[...], x, preferred_element_type=jnp.float32)
    h = h + bt_ref[...]
    mavg = mavg_ref[...]
    mu = jnp.dot(mavg, h, preferred_element_type=jnp.float32)
    msq = jnp.dot(mavg, h * h, preferred_element_type=jnp.float32)
    var = msq - mu * mu
    hn = (h - mu) * lax.rsqrt(var + 1e-5) * gt_ref[...] + lbt_ref[...]
    sil = hn * jax.nn.sigmoid(hn)
    bp = sil.shape[1]
    aa = aa_ref[...]
    ab = ab_ref[...]
    sa = sil[:fc, :] * jnp.broadcast_to(aa, (fc, bp))
    sb = sil[fc:, :] * jnp.broadcast_to(ab, (fc, bp))
    ones = jnp.ones((1, bp), jnp.float32)
    zpad = jnp.zeros((fc - 2, bp), jnp.float32)
    ta = jnp.concatenate([sa, aa, ones, zpad], axis=0)
    tb = jnp.concatenate([sb, ab, ones, zpad], axis=0)
    out_ref[0] = ta.T
    out_ref[1] = tb.T


def _edge_stage(edge_scalars, edge_attr, rad_w1, rad_b1, ln_g, ln_b,
                part, nparts):
    e, fc = edge_scalars.shape
    e2 = e // 2
    e2s = e2 // nparts
    es_t = edge_scalars.T
    a_row = edge_attr.T
    z = jnp.zeros((fc, fc), jnp.float32)
    w1t = rad_w1.T
    wt = jnp.block([[w1t, z], [z, w1t]])
    ones_avg = jnp.full((fc, fc), 1.0 / fc, jnp.float32)
    mavg = jnp.block([[ones_avg, z], [z, ones_avg]])
    bt = jnp.concatenate([rad_b1, rad_b1])[:, None]
    gt = jnp.concatenate([ln_g, ln_g])[:, None]
    lbt = jnp.concatenate([ln_b, ln_b])[:, None]
    bp = 3200
    assert e2s % bp == 0
    grid = e2s // bp
    gall = e2 // bp
    base = part * grid
    d2 = 2 * fc
    return pl.pallas_call(
        functools.partial(_edge_mlp_kernel, fc=fc),
        grid=(grid,),
        in_specs=[
            pl.BlockSpec((fc, bp), lambda i, b=base: (0, i + b)),
            pl.BlockSpec((fc, bp), lambda i, b=base, g=gall: (0, i + b + g)),
            pl.BlockSpec((1, bp), lambda i, b=base: (0, i + b)),
            pl.BlockSpec((1, bp), lambda i, b=base, g=gall: (0, i + b + g)),
            pl.BlockSpec((d2, d2), lambda i: (0, 0)),
            pl.BlockSpec((d2, 1), lambda i: (0, 0)),
            pl.BlockSpec((d2, 1), lambda i: (0, 0)),
            pl.BlockSpec((d2, 1), lambda i: (0, 0)),
            pl.BlockSpec((d2, d2), lambda i: (0, 0)),
        ],
        out_specs=pl.BlockSpec((2, bp, d2), lambda i: (0, i, 0)),
        out_shape=jax.ShapeDtypeStruct((2, e2s, d2), jnp.float32),
    )(es_t, es_t, a_row, a_row, wt, bt, gt, lbt, mavg)


def _sc_scatter_kernel(payload_hbm, dst_hbm, zeros_hbm, out_hbm,
                       pbuf, ibuf, pbuf_t, ibuf_t, acc, psem0, psem1,
                       isem0, isem1,
                       *, e2, ebase, per_tile, n_per_tile, chunk, tail):
    c = lax.axis_index("c")
    s = lax.axis_index("s")
    pltpu.sync_copy(zeros_hbm, acc.at[pl.ds(s * n_per_tile, n_per_tile), :])
    plsc.subcore_barrier()
    base = s * per_tile
    units = per_tile // chunk
    psems = (psem0, psem1)
    isems = (isem0, isem1)

    def start_load(u, slot):
        off = base + u * chunk
        pltpu.async_copy(payload_hbm.at[c, pl.ds(off, chunk), :],
                         pbuf.at[slot], psems[slot])
        pltpu.async_copy(dst_hbm.at[pl.ds(c * e2 + ebase + off, chunk)],
                         ibuf.at[slot], isems[slot])

    def do_scatter(slot):
        pltpu.make_async_copy(payload_hbm.at[c, pl.ds(0, chunk), :],
                              pbuf.at[slot], psems[slot]).wait()
        pltpu.make_async_copy(dst_hbm.at[pl.ds(0, chunk)], ibuf.at[slot],
                              isems[slot]).wait()
        pltpu.sync_copy(pbuf.at[slot], acc.at[ibuf.at[slot]], add=True)

    start_load(0, 0)

    def body(g, _):
        u = 2 * g
        start_load(u + 1, 1)
        do_scatter(0)

        @pl.when(u + 2 < units)
        def _():
            start_load(u + 2, 0)

        do_scatter(1)
        return 0

    lax.fori_loop(0, units // 2, body, 0)
    if units % 2:
        do_scatter(0)
    if tail:
        off = base + units * chunk
        pltpu.sync_copy(payload_hbm.at[c, pl.ds(off, tail), :], pbuf_t)
        pltpu.sync_copy(dst_hbm.at[pl.ds(c * e2 + ebase + off, tail)], ibuf_t)
        pltpu.sync_copy(pbuf_t, acc.at[ibuf_t], add=True)
    plsc.subcore_barrier()
    pltpu.sync_copy(acc.at[pl.ds(s * n_per_tile, n_per_tile), :],
                    out_hbm.at[c, pl.ds(s * n_per_tile, n_per_tile), :])


def _scatter_stage(payload, edge_dst, n, part, nparts):
    _, e2s, d2 = payload.shape
    e2 = e2s * nparts
    ebase = part * e2s
    assert e2s % 16 == 0
    per_tile = e2s // 16
    chunk = 128
    tail = per_tile % chunk
    n_pad = ((n + 127) // 128) * 128
    n_per_tile = n_pad // 16
    zeros = jnp.zeros((n_per_tile, d2), jnp.float32)
    mesh = plsc.VectorSubcoreMesh(core_axis_name="c", subcore_axis_name="s")
    scratch = [
        pltpu.VMEM((2, chunk, d2), jnp.float32),
        pltpu.VMEM((2, chunk), jnp.int32),
        pltpu.VMEM((max(tail, 8), d2), jnp.float32),
        pltpu.VMEM((max(tail, 8),), jnp.int32),
        pltpu.VMEM_SHARED((n_pad, d2), jnp.float32),
        pltpu.SemaphoreType.DMA,
        pltpu.SemaphoreType.DMA,
        pltpu.SemaphoreType.DMA,
        pltpu.SemaphoreType.DMA,
    ]
    kern = pl.kernel(
        functools.partial(_sc_scatter_kernel, e2=e2, ebase=ebase,
                          per_tile=per_tile, n_per_tile=n_per_tile,
                          chunk=chunk, tail=tail),
        out_type=jax.ShapeDtypeStruct((2, n_pad, d2), jnp.float32),
        mesh=mesh,
        scratch_types=scratch,
    )
    return kern(payload, edge_dst, zeros)


def _proj_kernel(parta_ref, partb_ref, wexp_ref, bexp_ref, w2_ref, roff_ref,
                 wproj_ref, bproj_ref, out_ref, *, fc, d, avg_agg):
    p = (parta_ref[0] + parta_ref[1]) + (partb_ref[0] + partb_ref[1])
    s = p[:, :fc]
    a = p[:, fc:fc + 1]
    cnt = p[:, fc + 1:fc + 2]
    v = wexp_ref[...][0] + bexp_ref[...]
    wc = jnp.dot(w2_ref[...] * v[None, :], wproj_ref[...],
                 preferred_element_type=jnp.float32)
    rvec = jnp.dot((v * roff_ref[...])[None, :], wproj_ref[...],
                   preferred_element_type=jnp.float32)
    scale = 1.0 / (math.sqrt(d) * math.sqrt(avg_agg))
    out = (jnp.dot(s, wc, preferred_element_type=jnp.float32) + a * rvec) * scale
    out = out + cnt * (bproj_ref[...][None, :] / math.sqrt(avg_agg))
    out_ref[...] = out


def _proj_stage(parta, partb, n, w_exp, b_exp, rad_w2, rad_offset, w_proj,
                b_proj, avg_agg):
    _, _, d = parta.shape
    fc = rad_w2.shape[0]
    bn = 2000
    assert n % bn == 0
    grid = n // bn
    return pl.pallas_call(
        functools.partial(_proj_kernel, fc=fc, d=d, avg_agg=avg_agg),
        grid=(grid,),
        in_specs=[
            pl.BlockSpec((2, bn, d), lambda i: (0, i, 0)),
            pl.BlockSpec((2, bn, d), lambda i: (0, i, 0)),
            pl.BlockSpec((1, d), lambda i: (0, 0)),
            pl.BlockSpec((d,), lambda i: (0,)),
            pl.BlockSpec((fc, d), lambda i: (0, 0)),
            pl.BlockSpec((d,), lambda i: (0,)),
            pl.BlockSpec((d, d), lambda i: (0, 0)),
            pl.BlockSpec((d,), lambda i: (0,)),
        ],
        out_specs=pl.BlockSpec((bn, d), lambda i: (i, 0)),
        out_shape=jax.ShapeDtypeStruct((n, d), jnp.float32),
    )(parta, partb, w_exp, b_exp, rad_w2, rad_offset, w_proj, b_proj)


def kernel(node_input, edge_attr, edge_scalars, edge_src, edge_dst, batch,
           w_exp, b_exp, rad_w1, rad_b1, ln_g, ln_b, rad_w2, rad_offset,
           w_proj, b_proj):
    n = node_input.shape[0]
    avg_agg = 32.0
    pay_a = _edge_stage(edge_scalars, edge_attr, rad_w1, rad_b1, ln_g, ln_b,
                        0, 2)
    part_a = _scatter_stage(pay_a, edge_dst, n, 0, 2)
    pay_b = _edge_stage(edge_scalars, edge_attr, rad_w1, rad_b1, ln_g, ln_b,
                        1, 2)
    part_b = _scatter_stage(pay_b, edge_dst, n, 1, 2)
    return _proj_stage(part_a, part_b, n, w_exp, b_exp, rad_w2, rad_offset,
                       w_proj, b_proj, avg_agg)
